# baseline (speedup 1.0000x reference)
import functools

import jax
import jax.numpy as jnp
from jax import lax
from jax.experimental import pallas as pl
from jax.experimental.pallas import tpu as pltpu
from jax.experimental.pallas import tpu_sc as plsc

N_ROWS = 320000
D_FEAT = 128
NUM_SEGMENTS = 10000

NC = 2
NS = 16
HALF = NUM_SEGMENTS // 2
DUMMY = HALF
ACC_ROWS = HALF + 16
BLK = 64
NBUF = 4
ZROWS = 24
ZSTRIPE = 312
ZLAST = ACC_ROWS - (NS - 1) * ZSTRIPE
WSTRIPE = 312
WLAST = HALF - (NS - 1) * WSTRIPE


def _sc_segment_sum(X, keysl2, r8arr):
    mesh = plsc.VectorSubcoreMesh(core_axis_name="c", subcore_axis_name="s")

    @functools.partial(
        pl.kernel,
        out_type=jax.ShapeDtypeStruct((NUM_SEGMENTS, D_FEAT), jnp.float32),
        mesh=mesh,
        scratch_types=[
            pltpu.VMEM((BLK, D_FEAT), jnp.float32),
            pltpu.VMEM((BLK, D_FEAT), jnp.float32),
            pltpu.VMEM((BLK, D_FEAT), jnp.float32),
            pltpu.VMEM((BLK, D_FEAT), jnp.float32),
            pltpu.VMEM((BLK,), jnp.int32),
            pltpu.VMEM((BLK,), jnp.int32),
            pltpu.VMEM((BLK,), jnp.int32),
            pltpu.VMEM((BLK,), jnp.int32),
            pltpu.VMEM((16, D_FEAT), jnp.float32),
            pltpu.VMEM((16,), jnp.int32),
            pltpu.VMEM((16,), jnp.int32),
            pltpu.VMEM((ZROWS, D_FEAT), jnp.float32),
            pltpu.VMEM_SHARED((ACC_ROWS, D_FEAT), jnp.float32),
            pltpu.SemaphoreType.DMA,
            pltpu.SemaphoreType.DMA,
            pltpu.SemaphoreType.DMA,
            pltpu.SemaphoreType.DMA,
            pltpu.SemaphoreType.DMA,
        ],
    )
    def k(x_hbm, keys_hbm, r8_hbm, out_hbm,
          xbuf_a, xbuf_b, xbuf_c, xbuf_d, kb_a, kb_b, kb_c, kb_d,
          sbuf, skbuf, rbuf, zbuf, acc,
          sem_a, sem_b, sem_c, sem_d, sem_z):
        c = lax.axis_index("c")
        s = lax.axis_index("s")

        pltpu.sync_copy(r8_hbm, rbuf)
        r8v = pl.multiple_of(rbuf[...][0], 8)

        coff = c * HALF
        start_c = jnp.where(c == 0, 0, r8v + 16)
        end_c = jnp.where(c == 0, r8v, N_ROWS)
        count_c = end_c - start_c
        per = (count_c // (NS * BLK)) * BLK
        base_s = pl.multiple_of(start_c + s * per, 8)
        count_last = count_c - (NS - 1) * per
        nfull = jnp.where(s < NS - 1, per // BLK, count_last // BLK)
        rem = count_last - (count_last // BLK) * BLK

        def start_load(i, xbuf, kbuf, sem):
            boff = pl.multiple_of(base_s + i * BLK, 8)
            pltpu.async_copy(x_hbm.at[pl.ds(boff, BLK)], xbuf, sem)
            pltpu.async_copy(keys_hbm.at[pl.ds(boff, BLK)], kbuf, sem)

        def wait_load(i, xbuf, kbuf, sem):
            boff = pl.multiple_of(base_s + i * BLK, 8)
            pltpu.make_async_copy(
                x_hbm.at[pl.ds(boff, BLK)], xbuf, sem).wait()
            pltpu.make_async_copy(
                keys_hbm.at[pl.ds(boff, BLK)], kbuf, sem).wait()

        def localize(kbuf):
            for q in range(BLK // 16):
                kbuf[pl.ds(q * 16, 16)] = kbuf[pl.ds(q * 16, 16)] - coff

        def scatter(xbuf, kbuf):
            pltpu.sync_copy(xbuf, acc.at[kbuf], add=True)

        bufs = ((xbuf_a, kb_a, sem_a), (xbuf_b, kb_b, sem_b),
                (xbuf_c, kb_c, sem_c), (xbuf_d, kb_d, sem_d))

        for b in range(NBUF):
            @pl.when(nfull > b)
            def _(b=b):
                start_load(b, *bufs[b])

        @pl.loop(0, ZROWS)
        def _(r):
            @pl.loop(0, D_FEAT, step=16)
            def _(col):
                zbuf[r, pl.ds(col, 16)] = jnp.zeros((16,), jnp.float32)

        zbase = pl.multiple_of(s * ZSTRIPE, 8)
        nz = jnp.where(s < NS - 1, ZSTRIPE // ZROWS, ZLAST // ZROWS)

        @pl.loop(0, ZLAST // ZROWS)
        def _(j):
            @pl.when(j < nz)
            def _():
                pltpu.async_copy(
                    zbuf, acc.at[pl.ds(zbase + j * ZROWS, ZROWS)], sem_z)

        @pl.loop(0, ZLAST // ZROWS)
        def _(j):
            @pl.when(j < nz)
            def _():
                pltpu.make_async_copy(
                    zbuf, acc.at[pl.ds(zbase + j * ZROWS, ZROWS)],
                    sem_z).wait()

        plsc.subcore_barrier()

        lanes = lax.iota(jnp.int32, 16)

        @pl.when(s == 0)
        def _():
            pltpu.sync_copy(x_hbm.at[pl.ds(r8v, 16)], sbuf)
            pltpu.sync_copy(keys_hbm.at[pl.ds(r8v, 16)], skbuf)
            v = skbuf[...] - coff
            valid = (v >= 0) & (v < HALF)
            skbuf[...] = jnp.where(valid, v, DUMMY + lanes)
            pltpu.sync_copy(sbuf, acc.at[skbuf], add=True)

        @pl.loop(0, (nfull + NBUF - 1) // NBUF)
        def _(g):
            for b in range(NBUF):
                i = NBUF * g + b

                @pl.when(i < nfull)
                def _(i=i, b=b):
                    wait_load(i, *bufs[b])
                    localize(bufs[b][1])
                    scatter(bufs[b][0], bufs[b][1])

                    @pl.when(i + NBUF < nfull)
                    def _():
                        start_load(i + NBUF, *bufs[b])

        @pl.when((s == NS - 1) & (rem > 0))
        def _():
            blockstart = pl.multiple_of(jnp.maximum(end_c - BLK, 0), 8)
            lo = base_s + (count_last // BLK) * BLK
            pltpu.sync_copy(x_hbm.at[pl.ds(blockstart, BLK)], xbuf_a)
            pltpu.sync_copy(keys_hbm.at[pl.ds(blockstart, BLK)], kb_a)
            for q in range(BLK // 16):
                pos = blockstart + q * 16 + lanes
                v = kb_a[pl.ds(q * 16, 16)] - coff
                valid = (pos >= lo) & (pos < end_c)
                kb_a[pl.ds(q * 16, 16)] = jnp.where(
                    valid, v, DUMMY + lanes)
            scatter(xbuf_a, kb_a)

        plsc.subcore_barrier()

        obase = pl.multiple_of(c * HALF + s * WSTRIPE, 8)

        @pl.when(s < NS - 1)
        def _():
            pltpu.sync_copy(
                acc.at[pl.ds(s * WSTRIPE, WSTRIPE)],
                out_hbm.at[pl.ds(obase, WSTRIPE)],
            )

        @pl.when(s == NS - 1)
        def _():
            pltpu.sync_copy(
                acc.at[pl.ds((NS - 1) * WSTRIPE, WLAST)],
                out_hbm.at[pl.ds(obase, WLAST)],
            )

    return k(X, keysl2, r8arr)


@jax.jit
def kernel(X, keys):
    keys32 = keys.astype(jnp.int32)
    r_star = jnp.searchsorted(keys32, jnp.int32(HALF)).astype(jnp.int32)
    r8 = jnp.minimum((r_star // 8) * 8, N_ROWS - 16)
    r8arr = jnp.full((16,), r8, jnp.int32)
    return _sc_segment_sum(X, keys32, r8arr)


# speedup vs baseline: 1.2394x; 1.2394x over previous
import functools

import jax
import jax.numpy as jnp
from jax import lax
from jax.experimental import pallas as pl
from jax.experimental.pallas import tpu as pltpu
from jax.experimental.pallas import tpu_sc as plsc

N_ROWS = 320000
D_FEAT = 128
NUM_SEGMENTS = 10000

NC = 2
NS = 16
NW = NC * NS
ROWS_PER_W = N_ROWS // NW
BLK = 40
NFULL = ROWS_PER_W // BLK
NBUF = 5
ZROWS = 16
WSTRIPE = 640
WLAST = NUM_SEGMENTS - (NS - 1) * WSTRIPE
OCHUNK = 80
NOCHUNK = NUM_SEGMENTS // OCHUNK
OITER = -(-NOCHUNK // NS)


def _sc_partial_sums(X, keys):
    mesh = plsc.VectorSubcoreMesh(core_axis_name="c", subcore_axis_name="s")

    @functools.partial(
        pl.kernel,
        out_type=jax.ShapeDtypeStruct((NC, NUM_SEGMENTS, D_FEAT), jnp.float32),
        mesh=mesh,
        scratch_types=[
            pltpu.VMEM((BLK, D_FEAT), jnp.float32),
            pltpu.VMEM((BLK, D_FEAT), jnp.float32),
            pltpu.VMEM((BLK, D_FEAT), jnp.float32),
            pltpu.VMEM((BLK, D_FEAT), jnp.float32),
            pltpu.VMEM((BLK, D_FEAT), jnp.float32),
            pltpu.VMEM((BLK,), jnp.int32),
            pltpu.VMEM((BLK,), jnp.int32),
            pltpu.VMEM((BLK,), jnp.int32),
            pltpu.VMEM((BLK,), jnp.int32),
            pltpu.VMEM((BLK,), jnp.int32),
            pltpu.VMEM((ZROWS, D_FEAT), jnp.float32),
            pltpu.VMEM_SHARED((NUM_SEGMENTS, D_FEAT), jnp.float32),
            pltpu.SemaphoreType.DMA,
            pltpu.SemaphoreType.DMA,
            pltpu.SemaphoreType.DMA,
            pltpu.SemaphoreType.DMA,
            pltpu.SemaphoreType.DMA,
            pltpu.SemaphoreType.DMA,
        ],
    )
    def k(x_hbm, keys_hbm, out_hbm, xbuf_a, xbuf_b, xbuf_c, xbuf_d, xbuf_e,
          kbuf_a, kbuf_b, kbuf_c, kbuf_d, kbuf_e,
          zbuf, acc, sem_a, sem_b, sem_c, sem_d, sem_e, sem_z):
        c = lax.axis_index("c")
        s = lax.axis_index("s")
        wid = c * NS + s

        base = wid * ROWS_PER_W

        def kslc(i):
            return keys_hbm.at[pl.ds(base + i * BLK, BLK)]

        def xslc(i):
            return x_hbm.at[pl.ds(base + i * BLK, BLK)]

        def start_load(i, xbuf, kbuf, sem):
            pltpu.async_copy(xslc(i), xbuf, sem)
            pltpu.async_copy(kslc(i), kbuf, sem)

        def wait_load(i, xbuf, kbuf, sem):
            pltpu.make_async_copy(xslc(i), xbuf, sem).wait()
            pltpu.make_async_copy(kslc(i), kbuf, sem).wait()

        bufs = ((xbuf_a, kbuf_a, sem_a), (xbuf_b, kbuf_b, sem_b),
                (xbuf_c, kbuf_c, sem_c), (xbuf_d, kbuf_d, sem_d),
                (xbuf_e, kbuf_e, sem_e))

        def refill(i, xbuf, kbuf, sem):
            @pl.when(i + NBUF < NFULL)
            def _():
                start_load(i + NBUF, xbuf, kbuf, sem)

        for b in range(NBUF):
            start_load(b, *bufs[b])

        @pl.loop(0, ZROWS)
        def _(r):
            @pl.loop(0, D_FEAT, step=16)
            def _(col):
                zbuf[r, pl.ds(col, 16)] = jnp.zeros((16,), jnp.float32)

        def each_zero_chunk(fn):
            @pl.loop(0, OITER)
            def _(j):
                chunk = s + NS * j

                @pl.when(chunk < NOCHUNK)
                def _():
                    @pl.loop(0, OCHUNK // ZROWS)
                    def _(j2):
                        fn(pl.ds(chunk * OCHUNK + j2 * ZROWS, ZROWS))

        each_zero_chunk(
            lambda d: pltpu.async_copy(zbuf, acc.at[d], sem_z))
        each_zero_chunk(
            lambda d: pltpu.make_async_copy(zbuf, acc.at[d], sem_z).wait())

        plsc.subcore_barrier()

        @pl.loop(0, NFULL // NBUF)
        def _(g):
            for b in range(NBUF):
                i = NBUF * g + b
                wait_load(i, *bufs[b])
                pltpu.sync_copy(bufs[b][0], acc.at[bufs[b][1]], add=True)
                refill(i, *bufs[b])

        plsc.subcore_barrier()

        @pl.when(s < NS - 1)
        def _():
            pltpu.sync_copy(
                acc.at[pl.ds(s * WSTRIPE, WSTRIPE)],
                out_hbm.at[c, pl.ds(s * WSTRIPE, WSTRIPE)],
            )

        @pl.when(s == NS - 1)
        def _():
            pltpu.sync_copy(
                acc.at[pl.ds((NS - 1) * WSTRIPE, WLAST)],
                out_hbm.at[c, pl.ds((NS - 1) * WSTRIPE, WLAST)],
            )

    return k(X, keys)


def _tc_combine(acc):
    def body(a_ref, b_ref, o_ref):
        o_ref[...] = a_ref[0] + b_ref[0]

    return pl.pallas_call(
        body,
        grid=(10,),
        in_specs=[
            pl.BlockSpec((1, 1000, D_FEAT), lambda i: (0, i, 0)),
            pl.BlockSpec((1, 1000, D_FEAT), lambda i: (1, i, 0)),
        ],
        out_specs=pl.BlockSpec((1000, D_FEAT), lambda i: (i, 0)),
        out_shape=jax.ShapeDtypeStruct((NUM_SEGMENTS, D_FEAT), jnp.float32),
    )(acc, acc)


@jax.jit
def kernel(X, keys):
    keys = keys.astype(jnp.int32)
    acc = _sc_partial_sums(X, keys)
    return _tc_combine(acc)
